# Initial kernel scaffold; baseline (speedup 1.0000x reference)
import jax
import jax.numpy as jnp
from jax.experimental import pallas as pl


def kernel(x, router_w, balance_bias, base_gate_w, base_up_w, base_down_w, experts):
    raise NotImplementedError("write your pallas kernel here")


# speedup vs baseline: 1.1212x; 1.1212x over previous
import functools

import jax
import jax.numpy as jnp
from jax import lax
from jax.experimental import pallas as pl
from jax.experimental.pallas import tpu as pltpu

_B, _T, _C = 2, 2048, 768
_N = _B * _T
_E = 8
_H = 1344
_BLK = 256
_NT = _N // _BLK + _E - 1
_P = _NT * _BLK
_TAU = 1.0
_BM = 512


def _router_kernel(x_ref, rw_ref, b_ref, lt_ref):
    lt = lax.dot_general(rw_ref[...].astype(jnp.bfloat16),
                         x_ref[...].astype(jnp.bfloat16),
                         (((0,), (1,)), ((), ())),
                         preferred_element_type=jnp.float32)
    lt_ref[...] = lt + b_ref[...]


def _base_kernel(x_ref, gw_ref, uw_ref, dw_ref, o_ref):
    x = x_ref[...]
    xb = x.astype(jnp.bfloat16)
    g = jnp.dot(xb, gw_ref[...], preferred_element_type=jnp.float32)
    u = jnp.dot(xb, uw_ref[...], preferred_element_type=jnp.float32)
    h = (g * lax.logistic(g) * u).astype(jnp.bfloat16)
    o_ref[...] = x + jnp.dot(h, dw_ref[...], preferred_element_type=jnp.float32)


def _expert_kernel(gmap_ref, xs_ref, gw_ref, uw_ref, dw_ref, ws_ref, ys_ref):
    xb = xs_ref[...].astype(jnp.bfloat16)
    g = jnp.dot(xb, gw_ref[0], preferred_element_type=jnp.float32)
    u = jnp.dot(xb, uw_ref[0], preferred_element_type=jnp.float32)
    h = (g * lax.logistic(g) * u).astype(jnp.bfloat16)
    y = jnp.dot(h, dw_ref[0], preferred_element_type=jnp.float32)
    ys_ref[...] = y * ws_ref[...]


def kernel(x, router_w, balance_bias, base_gate_w, base_up_w, base_down_w, experts):
    x_flat = x.reshape(_N, _C)

    logits_t = pl.pallas_call(
        _router_kernel,
        out_shape=jax.ShapeDtypeStruct((_E, _N), jnp.float32),
    )(x_flat, router_w, balance_bias.reshape(_E, 1))

    probs_t = jax.nn.sigmoid(logits_t.T / _TAU)
    e_id = jnp.argmax(probs_t, axis=-1)
    p = jnp.max(probs_t, axis=-1)
    w = p / jnp.maximum(p, 1e-6)
    onehot = (e_id[:, None] == jnp.arange(_E)[None, :]).astype(jnp.int32)
    rank = jnp.cumsum(onehot, axis=0)
    rank_t = jnp.take_along_axis(rank, e_id[:, None], axis=1)[:, 0] - 1
    counts = rank[-1]
    ptiles = (counts + _BLK - 1) // _BLK
    cum_tiles = jnp.cumsum(ptiles)
    row_off = (cum_tiles - ptiles) * _BLK
    dest = row_off[e_id] + rank_t
    ids_a = jnp.zeros((_P,), jnp.int32).at[dest].set(jnp.arange(_N, dtype=jnp.int32))
    ws = jnp.zeros((_P, 1), jnp.float32).at[dest, 0].set(w)
    gmap = jnp.clip(jnp.sum(jnp.arange(_NT)[:, None] >= cum_tiles[None, :],
                            axis=1), 0, _E - 1).astype(jnp.int32)

    xs = x_flat[ids_a]

    xb_out = pl.pallas_call(
        _base_kernel,
        grid=(_N // _BM,),
        in_specs=[
            pl.BlockSpec((_BM, _C), lambda i: (i, 0)),
            pl.BlockSpec(base_gate_w.shape, lambda i: (0, 0)),
            pl.BlockSpec(base_gate_w.shape, lambda i: (0, 0)),
            pl.BlockSpec(base_down_w.shape, lambda i: (0, 0)),
        ],
        out_specs=pl.BlockSpec((_BM, _C), lambda i: (i, 0)),
        out_shape=jax.ShapeDtypeStruct((_N, _C), jnp.float32),
    )(x_flat, base_gate_w.astype(jnp.bfloat16), base_up_w.astype(jnp.bfloat16),
      base_down_w.astype(jnp.bfloat16))

    gws = jnp.stack([jnp.pad(e['gate'], ((0, 0), (0, _H - e['gate'].shape[1])))
                     for e in experts]).astype(jnp.bfloat16)
    uws = jnp.stack([jnp.pad(e['up'], ((0, 0), (0, _H - e['up'].shape[1])))
                     for e in experts]).astype(jnp.bfloat16)
    dws = jnp.stack([jnp.pad(e['down'], ((0, _H - e['down'].shape[0]), (0, 0)))
                     for e in experts]).astype(jnp.bfloat16)

    ys = pl.pallas_call(
        _expert_kernel,
        grid_spec=pltpu.PrefetchScalarGridSpec(
            num_scalar_prefetch=1,
            grid=(_NT,),
            in_specs=[
                pl.BlockSpec((_BLK, _C), lambda i, g: (i, 0)),
                pl.BlockSpec((1, _C, _H), lambda i, g: (g[i], 0, 0)),
                pl.BlockSpec((1, _C, _H), lambda i, g: (g[i], 0, 0)),
                pl.BlockSpec((1, _H, _C), lambda i, g: (g[i], 0, 0)),
                pl.BlockSpec((_BLK, 1), lambda i, g: (i, 0)),
            ],
            out_specs=pl.BlockSpec((_BLK, _C), lambda i, g: (i, 0)),
        ),
        out_shape=jax.ShapeDtypeStruct((_P, _C), jnp.float32),
    )(gmap, xs, gws, uws, dws, ws)

    out = xb_out + ys[dest]
    return out.reshape(_B, _T, _C)
